# baseline (speedup 1.0000x reference)
import functools

import jax
import jax.numpy as jnp
from jax.experimental import pallas as pl
from jax.experimental.pallas import tpu as pltpu

_BN = 128
_PERSIST = 18


def _fused_kernel(nb, persist, n, hid,
                  a0_ref, a1_ref, x_ref, nrm_ref, watt_ref, batt_ref,
                  w_ref, b_ref, o_ref, s0_ref, s1_ref, rh_ref):
    lyr = pl.program_id(0)
    i = pl.program_id(1)
    bn = o_ref.shape[0]

    @pl.when((lyr == 0) & (i == 0))
    def _():
        rh_ref[0:n, 0:x_ref.shape[1]] = x_ref[...].astype(jnp.bfloat16)
        rh_ref[0:n, x_ref.shape[1]:hid] = jnp.zeros(
            (n, hid - x_ref.shape[1]), jnp.bfloat16)

    slot = jnp.where(i < persist, i, persist)
    off = slot * bn

    @pl.when((lyr == 0) | (i >= persist))
    def _():
        s0_ref[pl.ds(off, bn), :] = a0_ref[0].astype(jnp.bfloat16)
        s1_ref[pl.ds(off, bn), :] = a1_ref[0].astype(jnp.bfloat16)

    rhs = rh_ref[pl.ds(lyr * n, n), :]
    t0 = jnp.dot(s0_ref[pl.ds(off, bn), :], rhs,
                 preferred_element_type=jnp.float32)
    t1 = jnp.dot(s1_ref[pl.ds(off, bn), :], rhs,
                 preferred_element_type=jnp.float32)
    nrm = nrm_ref[...]
    t0 = t0 / nrm[:, 0:1]
    t1 = t1 / nrm[:, 1:2]
    watt = watt_ref[0]
    batt = batt_ref[0, 0, 0]
    l0 = jnp.dot(t0, watt, preferred_element_type=jnp.float32) + batt
    l1 = jnp.dot(t1, watt, preferred_element_type=jnp.float32) + batt
    mx = jnp.maximum(l0, l1)
    e0 = jnp.exp(l0 - mx)
    e1 = jnp.exp(l1 - mx)
    comb = (t0 * e0 + t1 * e1) / (e0 + e1)
    out = jnp.dot(comb, w_ref[0], preferred_element_type=jnp.float32)
    out = jnp.maximum(out + b_ref[0], 0.0)

    @pl.when(lyr == 0)
    def _():
        rh_ref[pl.ds(n + i * bn, bn), :] = out.astype(jnp.bfloat16)

    o_ref[...] = out[:, 0:o_ref.shape[1]]


def _encoder(x, motifs_all, nrm_t, watt, batt, w, b, *, interpret=False):
    n = x.shape[0]
    d_in = x.shape[1]
    hid = w.shape[2]
    d_out = d_in
    m = nrm_t.shape[1]
    bn, persist = _BN, _PERSIST
    nb = n // bn
    grid = (2, nb)

    def a_idx(mm):
        return lambda lyr, i: (
            mm, jnp.where((lyr == 1) & (i < persist), nb - 1, i), 0)

    return pl.pallas_call(
        functools.partial(_fused_kernel, nb, persist, n, hid),
        grid=grid,
        in_specs=[
            pl.BlockSpec((1, bn, n), a_idx(0)),
            pl.BlockSpec((1, bn, n), a_idx(1)),
            pl.BlockSpec((n, d_in), lambda lyr, i: (0, 0)),
            pl.BlockSpec((bn, m), lambda lyr, i: (i, 0)),
            pl.BlockSpec((1, hid, 1), lambda lyr, i: (lyr, 0, 0)),
            pl.BlockSpec((1, 1, 1), lambda lyr, i: (lyr, 0, 0)),
            pl.BlockSpec((1, hid, hid), lambda lyr, i: (lyr, 0, 0)),
            pl.BlockSpec((1, 1, hid), lambda lyr, i: (lyr, 0, 0)),
        ],
        out_specs=pl.BlockSpec((bn, d_out), lambda lyr, i: (lyr * i, 0)),
        out_shape=jax.ShapeDtypeStruct((n, d_out), jnp.float32),
        scratch_shapes=[
            pltpu.VMEM(((persist + 1) * bn, n), jnp.bfloat16),
            pltpu.VMEM(((persist + 1) * bn, n), jnp.bfloat16),
            pltpu.VMEM((2 * n, hid), jnp.bfloat16),
        ],
        compiler_params=pltpu.CompilerParams(
            dimension_semantics=("arbitrary", "arbitrary")),
        interpret=interpret,
    )(motifs_all, motifs_all, x, nrm_t, watt, batt, w, b)


@jax.jit
def kernel(x, motifs_all, motifs_num, w_att0, b_att0, W0, b0,
           w_att1, b_att1, W1, b1):
    hid = W0.shape[1]
    d_in = x.shape[1]
    nrm_t = motifs_num.T
    watt = jnp.stack([
        jnp.pad(w_att0, ((0, hid - d_in), (0, 0))),
        w_att1,
    ])
    batt = jnp.stack([b_att0.reshape(1, 1), b_att1.reshape(1, 1)])
    w = jnp.stack([
        jnp.pad(W0, ((0, hid - d_in), (0, 0))),
        jnp.pad(W1, ((0, 0), (0, hid - W1.shape[1]))),
    ])
    b = jnp.stack([
        b0.reshape(1, -1),
        jnp.pad(b1.reshape(1, -1), ((0, 0), (0, hid - b1.shape[0]))),
    ])
    return _encoder(x, motifs_all, nrm_t, watt, batt, w, b)


# speedup vs baseline: 1.0025x; 1.0025x over previous
import functools

import jax
import jax.numpy as jnp
from jax.experimental import pallas as pl
from jax.experimental.pallas import tpu as pltpu

_BN = 128
_PERSIST = 18


def _attend_project(t0, t1, nrm, watt, batt, w, bias):
    t0 = t0 / nrm[:, 0:1]
    t1 = t1 / nrm[:, 1:2]
    l0 = jnp.dot(t0, watt, preferred_element_type=jnp.float32) + batt
    l1 = jnp.dot(t1, watt, preferred_element_type=jnp.float32) + batt
    mx = jnp.maximum(l0, l1)
    e0 = jnp.exp(l0 - mx)
    e1 = jnp.exp(l1 - mx)
    comb = (t0 * e0 + t1 * e1) / (e0 + e1)
    out = jnp.dot(comb, w, preferred_element_type=jnp.float32)
    return jnp.maximum(out + bias, 0.0)


def _fused_kernel(persist, n,
                  a0_ref, a1_ref, x_ref, nrm_ref,
                  watt0_ref, batt0_ref, w0_ref, b0_ref,
                  watt1_ref, batt1_ref, w1_ref, b1_ref,
                  o_ref, s0_ref, s1_ref, xb_ref, hb_ref):
    lyr = pl.program_id(0)
    i = pl.program_id(1)
    bn = o_ref.shape[0]
    nrm = nrm_ref[...]

    @pl.when((lyr == 0) & (i == 0))
    def _():
        xb_ref[...] = x_ref[...].astype(jnp.bfloat16)

    @pl.when(lyr == 0)
    def _():
        a0b = a0_ref[0].astype(jnp.bfloat16)
        a1b = a1_ref[0].astype(jnp.bfloat16)

        @pl.when(i < persist)
        def _():
            s0_ref[pl.ds(i * bn, bn), :] = a0b
            s1_ref[pl.ds(i * bn, bn), :] = a1b

        xb = xb_ref[...]
        t0 = jnp.dot(a0b, xb, preferred_element_type=jnp.float32)
        t1 = jnp.dot(a1b, xb, preferred_element_type=jnp.float32)
        out = _attend_project(t0, t1, nrm, watt0_ref[...],
                              batt0_ref[0, 0], w0_ref[...], b0_ref[...])
        hb_ref[pl.ds(i * bn, bn), :] = out.astype(jnp.bfloat16)
        o_ref[...] = out[:, 0:o_ref.shape[1]]

    @pl.when(lyr == 1)
    def _():
        @pl.when(i >= persist)
        def _():
            s0_ref[pl.ds(persist * bn, bn), :] = a0_ref[0].astype(jnp.bfloat16)
            s1_ref[pl.ds(persist * bn, bn), :] = a1_ref[0].astype(jnp.bfloat16)

        off = jnp.where(i < persist, i, persist) * bn
        hb = hb_ref[...]
        t0 = jnp.dot(s0_ref[pl.ds(off, bn), :], hb,
                     preferred_element_type=jnp.float32)
        t1 = jnp.dot(s1_ref[pl.ds(off, bn), :], hb,
                     preferred_element_type=jnp.float32)
        o_ref[...] = _attend_project(t0, t1, nrm, watt1_ref[...],
                                     batt1_ref[0, 0], w1_ref[...],
                                     b1_ref[...])


def _encoder(x, motifs_all, nrm_t, w_att0, b_att0, W0, b0,
             w_att1, b_att1, W1, b1, *, interpret=False):
    n = x.shape[0]
    d_in = x.shape[1]
    hid = W0.shape[1]
    d_out = W1.shape[1]
    m = nrm_t.shape[1]
    bn, persist = _BN, _PERSIST
    nb = n // bn
    grid = (2, nb)

    def a_idx(mm):
        return lambda lyr, i: (
            mm, jnp.where((lyr == 1) & (i < persist), nb - 1, i), 0)

    cst = lambda lyr, i: (0, 0)
    return pl.pallas_call(
        functools.partial(_fused_kernel, persist, n),
        grid=grid,
        in_specs=[
            pl.BlockSpec((1, bn, n), a_idx(0)),
            pl.BlockSpec((1, bn, n), a_idx(1)),
            pl.BlockSpec((n, d_in), cst),
            pl.BlockSpec((bn, m), lambda lyr, i: (i, 0)),
            pl.BlockSpec((d_in, 1), cst),
            pl.BlockSpec((1, 1), cst),
            pl.BlockSpec((d_in, hid), cst),
            pl.BlockSpec((1, hid), cst),
            pl.BlockSpec((hid, 1), cst),
            pl.BlockSpec((1, 1), cst),
            pl.BlockSpec((hid, d_out), cst),
            pl.BlockSpec((1, d_out), cst),
        ],
        out_specs=pl.BlockSpec((bn, d_out), lambda lyr, i: (lyr * i, 0)),
        out_shape=jax.ShapeDtypeStruct((n, d_out), jnp.float32),
        scratch_shapes=[
            pltpu.VMEM(((persist + 1) * bn, n), jnp.bfloat16),
            pltpu.VMEM(((persist + 1) * bn, n), jnp.bfloat16),
            pltpu.VMEM((n, d_in), jnp.bfloat16),
            pltpu.VMEM((n, hid), jnp.bfloat16),
        ],
        compiler_params=pltpu.CompilerParams(
            dimension_semantics=("arbitrary", "arbitrary")),
        interpret=interpret,
    )(motifs_all, motifs_all, x, nrm_t,
      w_att0, b_att0, W0, b0, w_att1, b_att1, W1, b1)


@jax.jit
def kernel(x, motifs_all, motifs_num, w_att0, b_att0, W0, b0,
           w_att1, b_att1, W1, b1):
    nrm_t = motifs_num.T
    return _encoder(x, motifs_all, nrm_t,
                    w_att0, b_att0.reshape(1, 1), W0, b0.reshape(1, -1),
                    w_att1, b_att1.reshape(1, 1), W1, b1.reshape(1, -1))


# speedup vs baseline: 1.0093x; 1.0068x over previous
import functools

import jax
import jax.numpy as jnp
from jax.experimental import pallas as pl
from jax.experimental.pallas import tpu as pltpu

_BN = 128
_PERSIST = 18


def _attend_project(t0, t1, nrm, watt, batt, w, bias):
    t0 = t0 / nrm[:, 0:1]
    t1 = t1 / nrm[:, 1:2]
    l0 = jnp.dot(t0, watt, preferred_element_type=jnp.float32) + batt
    l1 = jnp.dot(t1, watt, preferred_element_type=jnp.float32) + batt
    mx = jnp.maximum(l0, l1)
    e0 = jnp.exp(l0 - mx)
    e1 = jnp.exp(l1 - mx)
    comb = (t0 * e0 + t1 * e1) / (e0 + e1)
    out = jnp.dot(comb, w, preferred_element_type=jnp.float32)
    return jnp.maximum(out + bias, 0.0)


def _fused_kernel(persist, n,
                  a0_ref, a1_ref, x_ref, nrm_ref,
                  watt0_ref, batt0_ref, w0_ref, b0_ref,
                  watt1_ref, batt1_ref, w1_ref, b1_ref,
                  o_ref, s0_ref, s1_ref, xb_ref, hb_ref):
    lyr = pl.program_id(0)
    i = pl.program_id(1)
    bn = o_ref.shape[0]
    nrm = nrm_ref[...]

    @pl.when((lyr == 0) & (i == 0))
    def _():
        xb_ref[...] = x_ref[...].astype(jnp.bfloat16)

    @pl.when(lyr == 0)
    def _():
        a0b = a0_ref[0].astype(jnp.bfloat16)
        a1b = a1_ref[0].astype(jnp.bfloat16)

        @pl.when(i < persist)
        def _():
            s0_ref[pl.ds(i * bn, bn), :] = a0b
            s1_ref[pl.ds(i * bn, bn), :] = a1b

        xb = xb_ref[...]
        t0 = jnp.dot(a0b, xb, preferred_element_type=jnp.float32)
        t1 = jnp.dot(a1b, xb, preferred_element_type=jnp.float32)
        out = _attend_project(t0, t1, nrm, watt0_ref[...],
                              batt0_ref[0, 0], w0_ref[...], b0_ref[...])
        hb_ref[pl.ds(i * bn, bn), :] = out.astype(jnp.bfloat16)
        o_ref[...] = out[:, 0:o_ref.shape[1]]

    @pl.when(lyr == 1)
    def _():
        @pl.when(i >= persist)
        def _():
            s0_ref[pl.ds(persist * bn, bn), :] = a0_ref[0].astype(jnp.bfloat16)
            s1_ref[pl.ds(persist * bn, bn), :] = a1_ref[0].astype(jnp.bfloat16)

        off = jnp.where(i < persist, i, persist) * bn
        hb = hb_ref[...]
        t0 = jnp.dot(s0_ref[pl.ds(off, bn), :], hb,
                     preferred_element_type=jnp.float32)
        t1 = jnp.dot(s1_ref[pl.ds(off, bn), :], hb,
                     preferred_element_type=jnp.float32)
        o_ref[...] = _attend_project(t0, t1, nrm, watt1_ref[...],
                                     batt1_ref[0, 0], w1_ref[...],
                                     b1_ref[...])


def _encoder(x, motifs_all, nrm_t, w_att0, b_att0, W0, b0,
             w_att1, b_att1, W1, b1, *, interpret=False):
    n = x.shape[0]
    d_in = x.shape[1]
    hid = W0.shape[1]
    d_out = W1.shape[1]
    m = nrm_t.shape[1]
    bn, persist = _BN, _PERSIST
    nb = n // bn
    grid = (2, nb)

    def a_idx(mm):
        return lambda lyr, i: (
            mm, jnp.where((lyr == 1) & (i < persist), nb - 1, i), 0)

    cst = lambda lyr, i: (0, 0)
    return pl.pallas_call(
        functools.partial(_fused_kernel, persist, n),
        grid=grid,
        in_specs=[
            pl.BlockSpec((1, bn, n), a_idx(0)),
            pl.BlockSpec((1, bn, n), a_idx(1)),
            pl.BlockSpec((n, d_in), cst),
            pl.BlockSpec((bn, m), lambda lyr, i: (i, 0)),
            pl.BlockSpec((d_in, 1), cst),
            pl.BlockSpec((1, 1), cst),
            pl.BlockSpec((d_in, hid), cst),
            pl.BlockSpec((1, hid), cst),
            pl.BlockSpec((hid, 1), cst),
            pl.BlockSpec((1, 1), cst),
            pl.BlockSpec((hid, d_out), cst),
            pl.BlockSpec((1, d_out), cst),
        ],
        out_specs=pl.BlockSpec((bn, d_out), lambda lyr, i: (i, 0)),
        out_shape=jax.ShapeDtypeStruct((n, d_out), jnp.float32),
        scratch_shapes=[
            pltpu.VMEM(((persist + 1) * bn, n), jnp.bfloat16),
            pltpu.VMEM(((persist + 1) * bn, n), jnp.bfloat16),
            pltpu.VMEM((n, d_in), jnp.bfloat16),
            pltpu.VMEM((n, hid), jnp.bfloat16),
        ],
        compiler_params=pltpu.CompilerParams(
            dimension_semantics=("arbitrary", "arbitrary")),
        interpret=interpret,
    )(motifs_all, motifs_all, x, nrm_t,
      w_att0, b_att0, W0, b0, w_att1, b_att1, W1, b1)


@jax.jit
def kernel(x, motifs_all, motifs_num, w_att0, b_att0, W0, b0,
           w_att1, b_att1, W1, b1):
    nrm_t = motifs_num.T
    return _encoder(x, motifs_all, nrm_t,
                    w_att0, b_att0.reshape(1, 1), W0, b0.reshape(1, -1),
                    w_att1, b_att1.reshape(1, 1), W1, b1.reshape(1, -1))


# speedup vs baseline: 1.0939x; 1.0839x over previous
import functools

import jax
import jax.numpy as jnp
from jax.experimental import pallas as pl
from jax.experimental.pallas import tpu as pltpu

_BN = 256
_SUB = 128
_PERSIST = 7


def _attend_project(t0, t1, nrm, watt, w, bias):
    t0 = t0 / nrm[:, 0:1]
    t1 = t1 / nrm[:, 1:2]
    u = t0 - t1
    d = jnp.dot(u, watt, preferred_element_type=jnp.float32)
    p = 1.0 / (1.0 + jnp.exp(-d))
    comb = t1 + p * u
    out = jnp.dot(comb, w, preferred_element_type=jnp.float32)
    return jnp.maximum(out + bias, 0.0)


def _fused_kernel(persist, n,
                  a0_ref, a1_ref, x_ref, nrm_ref,
                  watt0_ref, w0_ref, b0_ref,
                  watt1_ref, w1_ref, b1_ref,
                  o_ref, s0_ref, s1_ref, hb_ref):
    lyr = pl.program_id(0)
    i = pl.program_id(1)
    bn = o_ref.shape[0]
    sub = _SUB
    nsub = bn // sub

    @pl.when(lyr == 0)
    def _():
        xb = x_ref[...]
        watt = watt0_ref[...]
        w = w0_ref[...]
        bias = b0_ref[...]
        for h in range(nsub):
            lo = h * sub
            a0b = a0_ref[0, lo:lo + sub, :].astype(jnp.bfloat16)
            a1b = a1_ref[0, lo:lo + sub, :].astype(jnp.bfloat16)

            @pl.when(i < persist)
            def _():
                s0_ref[pl.ds(i * bn + lo, sub), :] = a0b
                s1_ref[pl.ds(i * bn + lo, sub), :] = a1b

            t0 = jnp.dot(a0b, xb, preferred_element_type=jnp.float32)
            t1 = jnp.dot(a1b, xb, preferred_element_type=jnp.float32)
            nrm = nrm_ref[lo:lo + sub]
            out = _attend_project(t0, t1, nrm, watt, w, bias)
            hb_ref[pl.ds(i * bn + lo, sub), :] = out.astype(jnp.bfloat16)
            o_ref[lo:lo + sub, :] = out[:, 0:o_ref.shape[1]]

    @pl.when(lyr == 1)
    def _():
        hb = hb_ref[...]
        watt = watt1_ref[...]
        w = w1_ref[...]
        bias = b1_ref[...]
        off = jnp.where(i < persist, i, persist) * bn
        for h in range(nsub):
            lo = h * sub

            @pl.when(i >= persist)
            def _():
                s0_ref[pl.ds(persist * bn + lo, sub), :] = (
                    a0_ref[0, lo:lo + sub, :].astype(jnp.bfloat16))
                s1_ref[pl.ds(persist * bn + lo, sub), :] = (
                    a1_ref[0, lo:lo + sub, :].astype(jnp.bfloat16))

            t0 = jnp.dot(s0_ref[pl.ds(off + lo, sub), :], hb,
                         preferred_element_type=jnp.float32)
            t1 = jnp.dot(s1_ref[pl.ds(off + lo, sub), :], hb,
                         preferred_element_type=jnp.float32)
            nrm = nrm_ref[lo:lo + sub]
            out = _attend_project(t0, t1, nrm, watt, w, bias)
            o_ref[lo:lo + sub, :] = out


def _encoder(x, motifs_all, nrm_t, w_att0, W0, b0,
             w_att1, W1, b1, *, interpret=False):
    n = x.shape[0]
    d_in = x.shape[1]
    hid = W0.shape[1]
    d_out = W1.shape[1]
    m = nrm_t.shape[1]
    bn, persist = _BN, _PERSIST
    nb = n // bn
    grid = (2, nb)

    def a_idx(mm):
        return lambda lyr, i: (
            mm, jnp.where((lyr == 1) & (i < persist), nb - 1, i), 0)

    cst = lambda lyr, i: (0, 0)
    return pl.pallas_call(
        functools.partial(_fused_kernel, persist, n),
        grid=grid,
        in_specs=[
            pl.BlockSpec((1, bn, n), a_idx(0)),
            pl.BlockSpec((1, bn, n), a_idx(1)),
            pl.BlockSpec((n, d_in), cst),
            pl.BlockSpec((bn, m), lambda lyr, i: (i, 0)),
            pl.BlockSpec((d_in, 1), cst),
            pl.BlockSpec((d_in, hid), cst),
            pl.BlockSpec((1, hid), cst),
            pl.BlockSpec((hid, 1), cst),
            pl.BlockSpec((hid, d_out), cst),
            pl.BlockSpec((1, d_out), cst),
        ],
        out_specs=pl.BlockSpec((bn, d_out), lambda lyr, i: (i, 0)),
        out_shape=jax.ShapeDtypeStruct((n, d_out), jnp.float32),
        scratch_shapes=[
            pltpu.VMEM(((persist + 1) * bn, n), jnp.bfloat16),
            pltpu.VMEM(((persist + 1) * bn, n), jnp.bfloat16),
            pltpu.VMEM((n, hid), jnp.bfloat16),
        ],
        compiler_params=pltpu.CompilerParams(
            dimension_semantics=("arbitrary", "arbitrary")),
        interpret=interpret,
    )(motifs_all, motifs_all, x, nrm_t,
      w_att0, W0, b0, w_att1, W1, b1)


@jax.jit
def kernel(x, motifs_all, motifs_num, w_att0, b_att0, W0, b0,
           w_att1, b_att1, W1, b1):
    del b_att0, b_att1
    nrm_t = motifs_num.T
    return _encoder(x.astype(jnp.bfloat16), motifs_all, nrm_t,
                    w_att0, W0, b0.reshape(1, -1),
                    w_att1, W1, b1.reshape(1, -1))


# speedup vs baseline: 1.1785x; 1.0773x over previous
import functools

import jax
import jax.numpy as jnp
from jax.experimental import pallas as pl
from jax.experimental.pallas import tpu as pltpu

_BN = 512
_SUB = 256


def _layer_kernel(a0_ref, a1_ref, x_ref, nrm_ref, watt_ref, w_ref, b_ref,
                  o_ref):
    x = x_ref[...]
    watt = watt_ref[...]
    w = w_ref[...]
    bias = b_ref[...]
    bn = o_ref.shape[0]
    for h in range(bn // _SUB):
        lo = h * _SUB
        t0 = jnp.dot(a0_ref[0, lo:lo + _SUB, :].astype(jnp.bfloat16), x,
                     preferred_element_type=jnp.float32)
        t1 = jnp.dot(a1_ref[0, lo:lo + _SUB, :].astype(jnp.bfloat16), x,
                     preferred_element_type=jnp.float32)
        nrm = nrm_ref[lo:lo + _SUB]
        t0 = t0 / nrm[:, 0:1]
        t1 = t1 / nrm[:, 1:2]
        u = t0 - t1
        d = jnp.dot(u, watt, preferred_element_type=jnp.float32)
        p = 1.0 / (1.0 + jnp.exp(-d))
        comb = t1 + p * u
        out = jnp.dot(comb, w, preferred_element_type=jnp.float32)
        out = jnp.maximum(out + bias, 0.0)
        o_ref[lo:lo + _SUB, :] = out.astype(o_ref.dtype)


def _layer(x, motifs_all, nrm_t, w_att, w, b, out_dtype, *,
           interpret=False):
    n = x.shape[0]
    d_in = x.shape[1]
    d_out = w.shape[1]
    m = nrm_t.shape[1]
    bn = _BN
    grid = (n // bn,)
    return pl.pallas_call(
        _layer_kernel,
        grid=grid,
        in_specs=[
            pl.BlockSpec((1, bn, n), lambda i: (0, i, 0)),
            pl.BlockSpec((1, bn, n), lambda i: (1, i, 0)),
            pl.BlockSpec((n, d_in), lambda i: (0, 0)),
            pl.BlockSpec((bn, m), lambda i: (i, 0)),
            pl.BlockSpec((d_in, 1), lambda i: (0, 0)),
            pl.BlockSpec((d_in, d_out), lambda i: (0, 0)),
            pl.BlockSpec((1, d_out), lambda i: (0, 0)),
        ],
        out_specs=pl.BlockSpec((bn, d_out), lambda i: (i, 0)),
        out_shape=jax.ShapeDtypeStruct((n, d_out), out_dtype),
        compiler_params=pltpu.CompilerParams(
            dimension_semantics=("arbitrary",)),
        interpret=interpret,
    )(motifs_all, motifs_all, x, nrm_t, w_att, w, b)


@jax.jit
def kernel(x, motifs_all, motifs_num, w_att0, b_att0, W0, b0,
           w_att1, b_att1, W1, b1):
    del b_att0, b_att1
    nrm_t = motifs_num.T
    h = _layer(x.astype(jnp.bfloat16), motifs_all, nrm_t,
               w_att0, W0, b0.reshape(1, -1), jnp.bfloat16)
    return _layer(h, motifs_all, nrm_t,
                  w_att1, W1, b1.reshape(1, -1), jnp.float32)
